# baseline (speedup 1.0000x reference)
import functools

import jax
import jax.numpy as jnp
from jax import lax
from jax.experimental import pallas as pl
from jax.experimental.pallas import tpu as pltpu
from jax.experimental.pallas import tpu_sc as plsc

_DOMS = (100, 100, 100)
_B = 16384

_NC = 2
_NS = 16
_NW = _NC * _NS
_BPW = _B // _NW
_CHUNK = 128
_NCHUNK = _BPW // _CHUNK
_XW = 3 * _BPW

_PHASE_RES = (0, 2, 1)
_PHASE_OFF = (0, 6, 11)
_S1 = _DOMS[1] * _DOMS[2]
_S2 = _DOMS[2]


def _sc_body(x_hbm, w_hbm, out_hbm, xv, idxv, rows, sem):
    wid = lax.axis_index("s") * _NC + lax.axis_index("c")
    base = wid * _BPW

    pltpu.sync_copy(x_hbm.at[pl.ds(base * 3, _XW)], xv.at[pl.ds(0, _XW)])

    lane = lax.iota(jnp.int32, 16)
    masks = [(lane % 3) == _PHASE_RES[r] for r in range(3)]

    for w in range(_XW // 16):
        a = xv[pl.ds(16 * w, 16)]
        b = xv[pl.ds(16 * w + 1, 16)]
        c = xv[pl.ds(16 * w + 2, 16)]
        t = a * _S1 + b * _S2 + c
        pos = 16 * (w // 3) + _PHASE_OFF[w % 3]
        plsc.store_compressed(idxv.at[pl.ds(pos, 16)], t, mask=masks[w % 3])

    copies = [
        pltpu.async_copy(
            w_hbm.at[idxv.at[pl.ds(j * _CHUNK, _CHUNK)]],
            rows.at[pl.ds(j * _CHUNK, _CHUNK)],
            sem,
        )
        for j in range(_NCHUNK)
    ]
    for c in copies:
        c.wait()

    pltpu.sync_copy(rows, out_hbm.at[pl.ds(base, _BPW)])


@functools.partial(jax.jit)
def _sc_gather(x_flat, w_flat):
    mesh = plsc.VectorSubcoreMesh(core_axis_name="c", subcore_axis_name="s")
    return pl.kernel(
        _sc_body,
        mesh=mesh,
        compiler_params=pltpu.CompilerParams(needs_layout_passes=False),
        out_type=jax.ShapeDtypeStruct((_B,), jnp.float32),
        scratch_types=[
            pltpu.VMEM((_XW + 8,), jnp.int32),
            pltpu.VMEM((_BPW + 16,), jnp.int32),
            pltpu.VMEM((_BPW,), jnp.float32),
            pltpu.SemaphoreType.DMA,
        ],
    )(x_flat, w_flat)


def kernel(x, W):
    x_flat = x.astype(jnp.int32).reshape(-1)
    w_flat = W.reshape(-1)
    return _sc_gather(x_flat, w_flat).reshape(_B, 1)


# speedup vs baseline: 1.3410x; 1.3410x over previous
import functools

import jax
import jax.numpy as jnp
from jax import lax
from jax.experimental import pallas as pl
from jax.experimental.pallas import tpu as pltpu
from jax.experimental.pallas import tpu_sc as plsc

_B = 16384

_NC = 2
_NS = 16
_NW = _NC * _NS
_BPW = _B // _NW
_CHUNK = 128
_NCHUNK = _BPW // _CHUNK
_L = 16
_D = 128


def _sc_body(x_hbm, w_hbm, out_hbm, xv, idxv, lanev, rows, outv, sem):
    wid = lax.axis_index("s") * _NC + lax.axis_index("c")
    base = wid * _BPW

    for d in range(3):
        pltpu.sync_copy(
            x_hbm.at[pl.ds(d * _B + base, _BPW)], xv.at[pl.ds(d * _BPW, _BPW)]
        )

    for g in range(_BPW // _L):
        i0 = xv[pl.ds(g * _L, _L)]
        i1 = xv[pl.ds(_BPW + g * _L, _L)]
        i2 = xv[pl.ds(2 * _BPW + g * _L, _L)]
        j, k = divmod(g, _CHUNK // _L)
        idxv[j, pl.ds(k * _L, _L)] = i0 * 100 + i1
        lanev[pl.ds(g * _L, _L)] = i2

    copies = [
        pltpu.async_copy(
            w_hbm.at[idxv.at[j]],
            rows.at[pl.ds(j * _CHUNK, _CHUNK), :],
            sem,
        )
        for j in range(_NCHUNK)
    ]
    for c in copies:
        c.wait()

    lane_iota = lax.iota(jnp.int32, _L)
    for g in range(_BPW // _L):
        rloc = lane_iota + g * _L
        vals = plsc.load_gather(rows, [rloc, lanev[pl.ds(g * _L, _L)]])
        outv[pl.ds(g * _L, _L)] = vals

    pltpu.sync_copy(outv, out_hbm.at[pl.ds(base, _BPW)])


@functools.partial(jax.jit)
def _sc_gather(x_flat, w2d):
    mesh = plsc.VectorSubcoreMesh(core_axis_name="c", subcore_axis_name="s")
    return pl.kernel(
        _sc_body,
        mesh=mesh,
        compiler_params=pltpu.CompilerParams(needs_layout_passes=False),
        out_type=jax.ShapeDtypeStruct((_B,), jnp.float32),
        scratch_types=[
            pltpu.VMEM((3 * _BPW,), jnp.int32),
            pltpu.VMEM((_NCHUNK, _CHUNK), jnp.int32),
            pltpu.VMEM((_BPW,), jnp.int32),
            pltpu.VMEM((_BPW, _D), jnp.float32),
            pltpu.VMEM((_BPW,), jnp.float32),
            pltpu.SemaphoreType.DMA,
        ],
    )(x_flat, w2d)


def kernel(x, W):
    x_flat = x.astype(jnp.int32).T.reshape(-1)
    w2d = jnp.pad(W.reshape(100 * 100, 100), ((0, 0), (0, _D - 100)))
    return _sc_gather(x_flat, w2d).reshape(_B, 1)
